# Initial kernel scaffold; baseline (speedup 1.0000x reference)
import jax
import jax.numpy as jnp
from jax.experimental import pallas as pl


def kernel(x):
    raise NotImplementedError("write your pallas kernel here")


# speedup vs baseline: 34.8764x; 34.8764x over previous
import functools

import jax
import jax.numpy as jnp
from jax import lax
from jax.experimental import pallas as pl
from jax.experimental.pallas import tpu as pltpu
from jax.experimental.pallas import tpu_sc as plsc
import numpy as np

N = 33554432
BINS = 2048
Q_MIN, Q_MAX = 0, 255
EPS = float(np.finfo(np.float32).eps)

XR, XC = 4096, 8192
BM = 256
GRID = XR // BM

NC, NS, L = 2, 16, 16
NW = NC * NS
PER_TILE = N // NW
CHUNK = 8192
NCHUNK = PER_TILE // CHUNK
PADB = 2065
UNROLL = 4


def _minmax_body(x_ref, min_ref, max_ref):
    i = pl.program_id(0)
    m = jnp.min(x_ref[...])
    mx = jnp.max(x_ref[...])

    @pl.when(i == 0)
    def _():
        min_ref[0, 0] = m
        max_ref[0, 0] = mx

    @pl.when(i != 0)
    def _():
        min_ref[0, 0] = jnp.minimum(min_ref[0, 0], m)
        max_ref[0, 0] = jnp.maximum(max_ref[0, 0], mx)


def _minmax(x2d):
    return pl.pallas_call(
        _minmax_body,
        grid=(GRID,),
        in_specs=[pl.BlockSpec((BM, XC), lambda i: (i, 0))],
        out_specs=[
            pl.BlockSpec((1, 1), lambda i: (0, 0), memory_space=pltpu.SMEM),
            pl.BlockSpec((1, 1), lambda i: (0, 0), memory_space=pltpu.SMEM),
        ],
        out_shape=[
            jax.ShapeDtypeStruct((1, 1), jnp.float32),
            jax.ShapeDtypeStruct((1, 1), jnp.float32),
        ],
    )(x2d)


def _hist_body(x_hbm, par_hbm, out_hbm, buf_a, buf_b, pvm, lh, hloc,
               sem_a, sem_b):
    wid = lax.axis_index("c") * NS + lax.axis_index("s")
    base = wid * PER_TILE

    pltpu.sync_copy(par_hbm, pvm)
    mn = pvm[pl.ds(0, L)]
    inv = pvm[pl.ds(L, L)]
    laneoff = lax.iota(jnp.int32, L) * PADB
    ones = jnp.full((L,), 1.0, jnp.float32)
    cap = jnp.full((L,), BINS - 1, jnp.int32)

    def zero_body(k, _):
        lh[pl.ds(k * L, L)] = jnp.zeros((L,), jnp.float32)
        return 0

    lax.fori_loop(0, (L * PADB) // L, zero_body, 0)

    def start(chunk, buf, sem):
        pltpu.make_async_copy(
            x_hbm.at[pl.ds(base + chunk * CHUNK, CHUNK)], buf, sem
        ).start()

    def wait(buf, sem):
        pltpu.make_async_copy(
            x_hbm.at[pl.ds(0, CHUNK)], buf, sem
        ).wait()

    def process(buf):
        def inner(j, _):
            for u in range(UNROLL):
                v = buf[pl.ds((j * UNROLL + u) * L, L)]
                t = (v - mn) * inv
                ix = jnp.minimum(t.astype(jnp.int32), cap) + laneoff
                plsc.addupdate_scatter(lh, [ix], ones)
            return 0

        lax.fori_loop(0, CHUNK // (L * UNROLL), inner, 0)

    start(0, buf_a, sem_a)
    start(1, buf_b, sem_b)

    def outer(i, _):
        c = i * 2
        wait(buf_a, sem_a)
        process(buf_a)
        start(c + 2, buf_a, sem_a)
        wait(buf_b, sem_b)
        process(buf_b)
        start(c + 3, buf_b, sem_b)
        return 0

    lax.fori_loop(0, NCHUNK // 2 - 1, outer, 0)
    wait(buf_a, sem_a)
    process(buf_a)
    wait(buf_b, sem_b)
    process(buf_b)

    def fold(stride):
        def body(v, _):
            o = v * L
            for l in range(0, 16, 2 * stride):
                if l + stride < 16:
                    a = lh[pl.ds(l * PADB + o, L)]
                    b = lh[pl.ds((l + stride) * PADB + o, L)]
                    lh[pl.ds(l * PADB + o, L)] = a + b
            return 0

        lax.fori_loop(0, BINS // L, body, 0)

    for stride in (1, 2, 4, 8):
        fold(stride)

    ovl_idx = lax.iota(jnp.int32, L) * PADB + BINS
    ovl = plsc.load_gather(lh, [ovl_idx])
    last = lh[pl.ds(BINS - L, L)]
    ovl_sum = jnp.sum(ovl)
    bump = jnp.where(lax.iota(jnp.int32, L) == L - 1, ovl_sum, 0.0)
    lh[pl.ds(BINS - L, L)] = last + bump

    def copy_out(v, _):
        hloc[pl.ds(v * L, L)] = lh[pl.ds(v * L, L)]
        return 0

    lax.fori_loop(0, BINS // L, copy_out, 0)
    pltpu.sync_copy(hloc, out_hbm.at[pl.ds(wid * BINS, BINS)])


def _hist_parts(x, par):
    mesh = plsc.VectorSubcoreMesh(core_axis_name="c", subcore_axis_name="s")
    f = pl.kernel(
        _hist_body,
        out_type=jax.ShapeDtypeStruct((NW * BINS,), jnp.float32),
        mesh=mesh,
        compiler_params=pltpu.CompilerParams(needs_layout_passes=False),
        scratch_types=[
            pltpu.VMEM((CHUNK,), jnp.float32),
            pltpu.VMEM((CHUNK,), jnp.float32),
            pltpu.VMEM((2 * L,), jnp.float32),
            pltpu.VMEM((L * PADB,), jnp.float32),
            pltpu.VMEM((BINS,), jnp.float32),
            pltpu.SemaphoreType.DMA,
            pltpu.SemaphoreType.DMA,
        ],
    )
    return f(x, par)


def _fq_body(s_ref, x_ref, o_ref):
    scale = s_ref[0]
    inv_scale = s_ref[1]
    zp = s_ref[2]
    x = x_ref[...]
    q = jnp.clip(jnp.round(x * inv_scale) + zp, float(Q_MIN), float(Q_MAX))
    o_ref[...] = (q - zp) * scale


def _fakequant(x2d, svec):
    return pl.pallas_call(
        _fq_body,
        grid=(GRID,),
        in_specs=[
            pl.BlockSpec(memory_space=pltpu.SMEM),
            pl.BlockSpec((BM, XC), lambda i: (i, 0)),
        ],
        out_specs=pl.BlockSpec((BM, XC), lambda i: (i, 0)),
        out_shape=jax.ShapeDtypeStruct((XR, XC), jnp.float32),
    )(svec, x2d)


def _merge_body(p_ref, h_ref):
    h_ref[...] = jnp.sum(p_ref[...], axis=0)


def _merge(parts):
    return pl.pallas_call(
        _merge_body,
        out_shape=jax.ShapeDtypeStruct((BINS,), jnp.float32),
    )(parts)


def kernel(x):
    x2d = x.reshape(XR, XC)
    mn_a, mx_a = _minmax(x2d)
    min_val = mn_a[0, 0]
    max_val = mx_a[0, 0]

    bin_width = (max_val - min_val) / BINS
    safe_w = jnp.maximum(bin_width, EPS)
    inv_w = 1.0 / safe_w

    min_val_neg = jnp.minimum(min_val, 0.0)
    max_val_pos = jnp.maximum(max_val, 0.0)
    scale = jnp.maximum((max_val_pos - min_val_neg) / float(Q_MAX - Q_MIN), EPS)
    zero_point = jnp.clip(
        Q_MIN - jnp.round(min_val_neg / scale), Q_MIN, Q_MAX
    )

    par = jnp.concatenate(
        [jnp.full((L,), min_val), jnp.full((L,), inv_w)]
    ).astype(jnp.float32)
    parts = _hist_parts(x, par)

    svec = jnp.stack([scale, 1.0 / scale, zero_point]).astype(jnp.float32)
    out = _fakequant(x2d, svec).reshape(N)

    histogram = _merge(parts.reshape(NW, BINS))
    return out, histogram, scale, zero_point.astype(jnp.int32)
